# Initial kernel scaffold; baseline (speedup 1.0000x reference)
import jax
import jax.numpy as jnp
from jax.experimental import pallas as pl


def kernel(x, edge_index, edge_attr, params):
    raise NotImplementedError("write your pallas kernel here")


# speedup vs baseline: 35.7097x; 35.7097x over previous
import functools

import jax
import jax.numpy as jnp
from jax import lax
from jax.experimental import pallas as pl
from jax.experimental.pallas import tpu as pltpu
from jax.experimental.pallas import tpu_sc as plsc

N = 10000
E = 320000
D_IN = 128
HID = 32
D_EDGE = 16
HEADS = 4
HEAD_DIM = 8
FC = 64
NUM_FC = 5
EPS = 1e-5

PW = 64
QW = 32
SW = 48

NC, NS = 2, 16
NW = NC * NS
LPR = 128
ROWS = E // LPR
RPW = 80
CH = 4

_HIGH = jax.lax.Precision.DEFAULT


def _bn_relu(x, g, b):
    mu = jnp.mean(x, axis=0, keepdims=True)
    var = jnp.mean((x - mu) * (x - mu), axis=0, keepdims=True)
    return jnp.maximum((x - mu) / jnp.sqrt(var + EPS) * g + b, 0.0)


def _pre_body(x_ref, g0_ref, b0_ref, w0_ref, bb0_ref, g1_ref, b1_ref, w1_ref,
              bb1_ref, wp_ref, bp_ref, wq_ref, h_ref, p_ref, q_ref):
    x = jnp.nan_to_num(x_ref[...])
    h = jnp.dot(_bn_relu(x, g0_ref[...], b0_ref[...]), w0_ref[...],
                preferred_element_type=jnp.float32, precision=_HIGH) + bb0_ref[...]
    h = h + jnp.dot(_bn_relu(h, g1_ref[...], b1_ref[...]), w1_ref[...],
                    preferred_element_type=jnp.float32, precision=_HIGH) + bb1_ref[...]
    h_ref[...] = h
    p_ref[...] = jnp.dot(h, wp_ref[...], preferred_element_type=jnp.float32,
                         precision=_HIGH) + bp_ref[...]
    q_ref[...] = jnp.dot(h, wq_ref[...], preferred_element_type=jnp.float32,
                         precision=_HIGH)


def _pre_call(x, g0, b0, w0, bb0, g1, b1, w1, bb1, wp, bp, wq):
    return pl.pallas_call(
        _pre_body,
        out_shape=(
            jax.ShapeDtypeStruct((N, HID), jnp.float32),
            jax.ShapeDtypeStruct((N, PW), jnp.float32),
            jax.ShapeDtypeStruct((N, QW), jnp.float32),
        ),
    )(x, g0, b0, w0, bb0, g1, b1, w1, bb1, wp, bp, wq)


BE = 4000


def _edge_body(ps_ref, qd_ref, e_ref, wee_ref, wae_ref, enew_ref, scat_ref):
    ps = ps_ref[...]
    qd = qd_ref[...]
    ev = jnp.nan_to_num(e_ref[...])
    enew = jnp.maximum(
        ps[:, :16] + qd[:, :16]
        + jnp.dot(ev, wee_ref[...], preferred_element_type=jnp.float32,
                  precision=_HIGH), 0.0)
    z = ps[:, 16:20] + qd[:, 16:20] + jnp.dot(
        enew, wae_ref[...], preferred_element_type=jnp.float32, precision=_HIGH)
    z = jnp.where(z >= 0.0, z, 0.2 * z)
    ex = jnp.exp(z)
    parts = [ps[:, 24 + h * 8:32 + h * 8] * ex[:, h:h + 1] for h in range(HEADS)]
    parts.append(ex)
    parts.append(jnp.zeros((BE, SW - HID - HEADS), jnp.float32))
    enew_ref[...] = enew
    scat_ref[...] = jnp.concatenate(parts, axis=1)


def _edge_call(ps, qd, e, wee, wae):
    grid = (E // BE,)
    return pl.pallas_call(
        _edge_body,
        grid=grid,
        in_specs=[
            pl.BlockSpec((BE, PW), lambda i: (i, 0)),
            pl.BlockSpec((BE, QW), lambda i: (i, 0)),
            pl.BlockSpec((BE, D_EDGE), lambda i: (i, 0)),
            pl.BlockSpec((D_EDGE, D_EDGE), lambda i: (0, 0)),
            pl.BlockSpec((D_EDGE, HEADS), lambda i: (0, 0)),
        ],
        out_specs=[
            pl.BlockSpec((BE, D_EDGE), lambda i: (i, 0)),
            pl.BlockSpec((BE, SW), lambda i: (i, 0)),
        ],
        out_shape=[
            jax.ShapeDtypeStruct((E, D_EDGE), jnp.float32),
            jax.ShapeDtypeStruct((E, SW), jnp.float32),
        ],
    )(ps, qd, e, wee, wae)


def _agg(nd0, nd1):
    num = nd0[:, :HID] + nd1[:, :HID]
    den = nd0[:, HID:HID + HEADS] + nd1[:, HID:HID + HEADS]
    return jnp.concatenate(
        [num[:, h * 8:(h + 1) * 8] / (den[:, h:h + 1] + 1e-16)
         for h in range(HEADS)], axis=1)


def _update_body(h_ref, nd_ref, wp_ref, bp_ref, wq_ref, h_out, p_out, q_out):
    hn = jnp.maximum(h_ref[...] + _agg(nd_ref[0], nd_ref[1]), 0.0)
    h_out[...] = hn
    p_out[...] = jnp.dot(hn, wp_ref[...], preferred_element_type=jnp.float32,
                         precision=_HIGH) + bp_ref[...]
    q_out[...] = jnp.dot(hn, wq_ref[...], preferred_element_type=jnp.float32,
                         precision=_HIGH)


def _update_call(h, nd, wp, bp, wq):
    return pl.pallas_call(
        _update_body,
        out_shape=(
            jax.ShapeDtypeStruct((N, HID), jnp.float32),
            jax.ShapeDtypeStruct((N, PW), jnp.float32),
            jax.ShapeDtypeStruct((N, QW), jnp.float32),
        ),
    )(h, nd, wp, bp, wq)


def _final_body(h_ref, nd_ref, *refs):
    out_ref = refs[-1]
    h = jnp.maximum(h_ref[...] + _agg(nd_ref[0], nd_ref[1]), 0.0)
    for i in range(NUM_FC):
        g, b, w, fb = refs[4 * i:4 * i + 4]
        h = jnp.dot(_bn_relu(h, g[...], b[...]), w[...],
                    preferred_element_type=jnp.float32, precision=_HIGH) + fb[...]
    out_ref[...] = h


def _final_call(h, nd, fc_args):
    return pl.pallas_call(
        _final_body,
        out_shape=jax.ShapeDtypeStruct((N, FC), jnp.float32),
    )(h, nd, *fc_args)


@functools.cache
def _mesh():
    return plsc.VectorSubcoreMesh(core_axis_name="c", subcore_axis_name="s",
                                  num_cores=NC, num_subcores=NS)


@functools.cache
def _gather_kernel():
    return functools.partial(
        pl.kernel,
        out_type=(
            jax.ShapeDtypeStruct((E, PW), jnp.float32),
            jax.ShapeDtypeStruct((E, QW), jnp.float32),
        ),
        mesh=_mesh(),
        scratch_types=[
            pltpu.VMEM((CH, LPR), jnp.int32),
            pltpu.VMEM((CH, LPR), jnp.int32),
            pltpu.VMEM((CH * LPR, PW), jnp.float32),
            pltpu.VMEM((CH * LPR, QW), jnp.float32),
            pltpu.SemaphoreType.DMA,
        ],
        compiler_params=pltpu.CompilerParams(use_tc_tiling_on_sc=False),
    )(_gather_body)


def _gather_body(src_hbm, dst_hbm, p_hbm, q_hbm, ps_out, qd_out,
                 idxs_v, idxd_v, prow_v, qrow_v, sem):
    w = lax.axis_index("s") * NC + lax.axis_index("c")

    def chunk(jj, _):
        row0 = w * RPW + jj * CH

        @pl.when(row0 < ROWS)
        def _():
            pltpu.sync_copy(src_hbm.at[pl.ds(row0, CH)], idxs_v)
            pltpu.sync_copy(dst_hbm.at[pl.ds(row0, CH)], idxd_v)
            cps = []
            for j in range(CH):
                cps.append(pltpu.async_copy(
                    p_hbm.at[idxs_v.at[j]],
                    prow_v.at[pl.ds(j * LPR, LPR)], sem))
                cps.append(pltpu.async_copy(
                    q_hbm.at[idxd_v.at[j]],
                    qrow_v.at[pl.ds(j * LPR, LPR)], sem))
            for cp in cps:
                cp.wait()
            pltpu.sync_copy(prow_v, ps_out.at[pl.ds(row0 * LPR, CH * LPR)])
            pltpu.sync_copy(qrow_v, qd_out.at[pl.ds(row0 * LPR, CH * LPR)])

        return ()

    lax.fori_loop(0, RPW // CH, chunk, ())


def _gather_call(src, dst, p_tab, q_tab):
    return _gather_kernel()(src, dst, p_tab, q_tab)


NPT = N // NS


@functools.cache
def _scatter_kernel():
    return functools.partial(
        pl.kernel,
        out_type=jax.ShapeDtypeStruct((NC, N, SW), jnp.float32),
        mesh=_mesh(),
        scratch_types=[
            pltpu.VMEM_SHARED((N, SW), jnp.float32),
            pltpu.VMEM((CH, LPR), jnp.int32),
            pltpu.VMEM((CH * LPR, SW), jnp.float32),
        ],
        compiler_params=pltpu.CompilerParams(use_tc_tiling_on_sc=False),
    )(_scatter_body)


def _scatter_body(scat_hbm, dst_hbm, zeros_hbm, nd_out, acc, idx_v, vals_v):
    cid = lax.axis_index("c")
    sid = lax.axis_index("s")
    w = sid * NC + cid

    @pl.when(sid == 0)
    def _():
        pltpu.sync_copy(zeros_hbm, acc)

    plsc.subcore_barrier()

    def chunk(jj, _):
        row0 = w * RPW + jj * CH

        @pl.when(row0 < ROWS)
        def _():
            pltpu.sync_copy(dst_hbm.at[pl.ds(row0, CH)], idx_v)
            pltpu.sync_copy(scat_hbm.at[pl.ds(row0 * LPR, CH * LPR)], vals_v)
            for j in range(CH):
                pltpu.sync_copy(vals_v.at[pl.ds(j * LPR, LPR)],
                                acc.at[idx_v.at[j]], add=True)

        return ()

    lax.fori_loop(0, RPW // CH, chunk, ())
    plsc.subcore_barrier()
    pltpu.sync_copy(acc.at[pl.ds(sid * NPT, NPT)],
                    nd_out.at[cid].at[pl.ds(sid * NPT, NPT)])


def _scatter_call(scat, dst, zeros_acc):
    return _scatter_kernel()(scat, dst, zeros_acc)


def _fold_conv(cp):
    we, wa, wm = cp['We'], cp['Wa'], cp['Wm']
    z = jnp.zeros
    wp = jnp.concatenate([we[:HID], wa[:HID], z((HID, 4), jnp.float32),
                          wm, z((HID, 8), jnp.float32)], axis=1)
    bp = jnp.concatenate([cp['be'], cp['ba'], z((4,), jnp.float32),
                          cp['bm'], z((8,), jnp.float32)])[None, :]
    wq = jnp.concatenate([we[HID:2 * HID], wa[HID:2 * HID],
                          z((HID, 12), jnp.float32)], axis=1)
    return wp, bp, wq, we[2 * HID:], wa[2 * HID:]


def kernel(x, edge_index, edge_attr, params):
    p = params
    src = edge_index[0].reshape(ROWS, LPR)
    dst = edge_index[1].reshape(ROWS, LPR)
    zeros_acc = jnp.zeros((N, SW), jnp.float32)

    folded = [_fold_conv(p['conv0']), _fold_conv(p['conv1'])]
    r2 = lambda a: a[None, :]

    h, P, Q = _pre_call(
        x, r2(p['bn0_g']), r2(p['bn0_b']), p['W0'], r2(p['b0']),
        r2(p['bn1_g']), r2(p['bn1_b']), p['W1'], r2(p['b1']),
        folded[0][0], folded[0][1], folded[0][2])

    e = edge_attr
    for layer in range(5):
        wee, wae = folded[0 if layer == 0 else 1][3:]
        ps, qd = _gather_call(src, dst, P, Q)
        e, scat = _edge_call(ps, qd, e, wee, wae)
        nd = _scatter_call(scat, dst, zeros_acc)
        if layer < 4:
            nwp, nbp, nwq = folded[1][:3]
            h, P, Q = _update_call(h, nd, nwp, nbp, nwq)

    fc_args = []
    for i in range(NUM_FC):
        fc_args += [r2(p['fc%d_g' % i]), r2(p['fc%d_b' % i]),
                    p['fcW%d' % i], r2(p['fcb%d' % i])]
    return _final_call(h, nd, fc_args)


# speedup vs baseline: 38.1822x; 1.0692x over previous
import functools

import jax
import jax.numpy as jnp
from jax import lax
from jax.experimental import pallas as pl
from jax.experimental.pallas import tpu as pltpu
from jax.experimental.pallas import tpu_sc as plsc

N = 10000
E = 320000
D_IN = 128
HID = 32
D_EDGE = 16
HEADS = 4
HEAD_DIM = 8
FC = 64
NUM_FC = 5
EPS = 1e-5

PW = 64
QW = 64
SW = 64

NC, NS = 2, 16
NW = NC * NS
LPR = 128
ROWS = E // LPR
RPW = 80
CH = 4

_HIGH = jax.lax.Precision.DEFAULT


def _bn_relu(x, g, b):
    mu = jnp.mean(x, axis=0, keepdims=True)
    var = jnp.mean((x - mu) * (x - mu), axis=0, keepdims=True)
    return jnp.maximum((x - mu) / jnp.sqrt(var + EPS) * g + b, 0.0)


def _pre_body(x_ref, g0_ref, b0_ref, w0_ref, bb0_ref, g1_ref, b1_ref, w1_ref,
              bb1_ref, wp_ref, bp_ref, wq_ref, h_ref, p_ref, q_ref):
    x = jnp.nan_to_num(x_ref[...])
    h = jnp.dot(_bn_relu(x, g0_ref[...], b0_ref[...]), w0_ref[...],
                preferred_element_type=jnp.float32, precision=_HIGH) + bb0_ref[...]
    h = h + jnp.dot(_bn_relu(h, g1_ref[...], b1_ref[...]), w1_ref[...],
                    preferred_element_type=jnp.float32, precision=_HIGH) + bb1_ref[...]
    h_ref[...] = h
    p_ref[...] = jnp.dot(h, wp_ref[...], preferred_element_type=jnp.float32,
                         precision=_HIGH) + bp_ref[...]
    q_ref[...] = jnp.dot(h, wq_ref[...], preferred_element_type=jnp.float32,
                         precision=_HIGH)


def _pre_call(x, g0, b0, w0, bb0, g1, b1, w1, bb1, wp, bp, wq):
    return pl.pallas_call(
        _pre_body,
        out_shape=(
            jax.ShapeDtypeStruct((N, HID), jnp.float32),
            jax.ShapeDtypeStruct((N, PW), jnp.float32),
            jax.ShapeDtypeStruct((N, QW), jnp.float32),
        ),
    )(x, g0, b0, w0, bb0, g1, b1, w1, bb1, wp, bp, wq)


B2 = 2000
E2 = E // 2


def _edge_math(ps, qd, ev, wee, wae):
    enew = jnp.maximum(
        ps[:, :16] + qd[:, :16]
        + jnp.dot(ev, wee, preferred_element_type=jnp.float32,
                  precision=_HIGH), 0.0)
    z = ps[:, 16:20] + qd[:, 16:20] + jnp.dot(
        enew, wae, preferred_element_type=jnp.float32, precision=_HIGH)
    z = jnp.where(z >= 0.0, z, 0.2 * z)
    ex = jnp.exp(z)
    parts = [ps[:, 24 + h * 8:32 + h * 8] * ex[:, h:h + 1] for h in range(HEADS)]
    parts.append(ex)
    parts.append(jnp.zeros((ps.shape[0], 4), jnp.float32))
    parts.append(enew)
    parts.append(jnp.zeros((ps.shape[0], 8), jnp.float32))
    return jnp.concatenate(parts, axis=1)


def _edge_body(ps_ref, qd_ref, ep_ref, wee_ref, wae_ref, out_ref):
    ps, qd, ep = ps_ref[...], qd_ref[...], ep_ref[...]
    wee, wae = wee_ref[...], wae_ref[...]
    out_l = _edge_math(ps[:, :64], qd[:, :64], ep[:, 40:56], wee, wae)
    out_r = _edge_math(ps[:, 64:], qd[:, 64:], ep[:, 104:120], wee, wae)
    out_ref[...] = jnp.concatenate([out_l, out_r], axis=1)


def _edge0_body(ps_ref, qd_ref, eev_ref, eod_ref, wee_ref, wae_ref, out_ref):
    ps, qd = ps_ref[...], qd_ref[...]
    wee, wae = wee_ref[...], wae_ref[...]
    out_l = _edge_math(ps[:, :64], qd[:, :64], jnp.nan_to_num(eev_ref[...]),
                       wee, wae)
    out_r = _edge_math(ps[:, 64:], qd[:, 64:], jnp.nan_to_num(eod_ref[...]),
                       wee, wae)
    out_ref[...] = jnp.concatenate([out_l, out_r], axis=1)


def _edge_call(ps2, qd2, eprev2, wee, wae):
    return pl.pallas_call(
        _edge_body,
        grid=(E2 // B2,),
        in_specs=[
            pl.BlockSpec((B2, 128), lambda i: (i, 0)),
            pl.BlockSpec((B2, 128), lambda i: (i, 0)),
            pl.BlockSpec((B2, 128), lambda i: (i, 0)),
            pl.BlockSpec((D_EDGE, D_EDGE), lambda i: (0, 0)),
            pl.BlockSpec((D_EDGE, HEADS), lambda i: (0, 0)),
        ],
        out_specs=pl.BlockSpec((B2, 128), lambda i: (i, 0)),
        out_shape=jax.ShapeDtypeStruct((E2, 128), jnp.float32),
    )(ps2, qd2, eprev2, wee, wae)


def _edge0_call(ps2, qd2, e_ev, e_od, wee, wae):
    return pl.pallas_call(
        _edge0_body,
        grid=(E2 // B2,),
        in_specs=[
            pl.BlockSpec((B2, 128), lambda i: (i, 0)),
            pl.BlockSpec((B2, 128), lambda i: (i, 0)),
            pl.BlockSpec((B2, D_EDGE), lambda i: (i, 0)),
            pl.BlockSpec((B2, D_EDGE), lambda i: (i, 0)),
            pl.BlockSpec((D_EDGE, D_EDGE), lambda i: (0, 0)),
            pl.BlockSpec((D_EDGE, HEADS), lambda i: (0, 0)),
        ],
        out_specs=pl.BlockSpec((B2, 128), lambda i: (i, 0)),
        out_shape=jax.ShapeDtypeStruct((E2, 128), jnp.float32),
    )(ps2, qd2, e_ev, e_od, wee, wae)


def _agg(nd0, nd1):
    num = nd0[:, :HID] + nd1[:, :HID]
    den = nd0[:, HID:HID + HEADS] + nd1[:, HID:HID + HEADS]
    return jnp.concatenate(
        [num[:, h * 8:(h + 1) * 8] / (den[:, h:h + 1] + 1e-16)
         for h in range(HEADS)], axis=1)


def _update_body(h_ref, nd_ref, wp_ref, bp_ref, wq_ref, h_out, p_out, q_out):
    hn = jnp.maximum(h_ref[...] + _agg(nd_ref[0], nd_ref[1]), 0.0)
    h_out[...] = hn
    p_out[...] = jnp.dot(hn, wp_ref[...], preferred_element_type=jnp.float32,
                         precision=_HIGH) + bp_ref[...]
    q_out[...] = jnp.dot(hn, wq_ref[...], preferred_element_type=jnp.float32,
                         precision=_HIGH)


def _update_call(h, nd, wp, bp, wq):
    return pl.pallas_call(
        _update_body,
        out_shape=(
            jax.ShapeDtypeStruct((N, HID), jnp.float32),
            jax.ShapeDtypeStruct((N, PW), jnp.float32),
            jax.ShapeDtypeStruct((N, QW), jnp.float32),
        ),
    )(h, nd, wp, bp, wq)


def _final_body(h_ref, nd_ref, *refs):
    out_ref = refs[-1]
    h = jnp.maximum(h_ref[...] + _agg(nd_ref[0], nd_ref[1]), 0.0)
    for i in range(NUM_FC):
        g, b, w, fb = refs[4 * i:4 * i + 4]
        h = jnp.dot(_bn_relu(h, g[...], b[...]), w[...],
                    preferred_element_type=jnp.float32, precision=_HIGH) + fb[...]
    out_ref[...] = h


def _final_call(h, nd, fc_args):
    return pl.pallas_call(
        _final_body,
        out_shape=jax.ShapeDtypeStruct((N, FC), jnp.float32),
    )(h, nd, *fc_args)


@functools.cache
def _mesh():
    return plsc.VectorSubcoreMesh(core_axis_name="c", subcore_axis_name="s",
                                  num_cores=NC, num_subcores=NS)


@functools.cache
def _gather_kernel():
    return functools.partial(
        pl.kernel,
        out_type=(
            jax.ShapeDtypeStruct((E, PW), jnp.float32),
            jax.ShapeDtypeStruct((E, QW), jnp.float32),
        ),
        mesh=_mesh(),
        scratch_types=[
            pltpu.VMEM((CH, LPR), jnp.int32),
            pltpu.VMEM((CH, LPR), jnp.int32),
            pltpu.VMEM((CH * LPR, PW), jnp.float32),
            pltpu.VMEM((CH * LPR, QW), jnp.float32),
            pltpu.SemaphoreType.DMA,
        ],
        compiler_params=pltpu.CompilerParams(use_tc_tiling_on_sc=False),
    )(_gather_body)


def _gather_body(src_hbm, dst_hbm, p_hbm, q_hbm, ps_out, qd_out,
                 idxs_v, idxd_v, prow_v, qrow_v, sem):
    w = lax.axis_index("s") * NC + lax.axis_index("c")

    def chunk(jj, _):
        row0 = w * RPW + jj * CH

        @pl.when(row0 < ROWS)
        def _():
            pltpu.sync_copy(src_hbm.at[pl.ds(row0, CH)], idxs_v)
            pltpu.sync_copy(dst_hbm.at[pl.ds(row0, CH)], idxd_v)
            cps = []
            for j in range(CH):
                cps.append(pltpu.async_copy(
                    p_hbm.at[idxs_v.at[j]],
                    prow_v.at[pl.ds(j * LPR, LPR)], sem))
                cps.append(pltpu.async_copy(
                    q_hbm.at[idxd_v.at[j]],
                    qrow_v.at[pl.ds(j * LPR, LPR)], sem))
            for cp in cps:
                cp.wait()
            pltpu.sync_copy(prow_v, ps_out.at[pl.ds(row0 * LPR, CH * LPR)])
            pltpu.sync_copy(qrow_v, qd_out.at[pl.ds(row0 * LPR, CH * LPR)])

        return ()

    lax.fori_loop(0, RPW // CH, chunk, ())


def _gather_call(src, dst, p_tab, q_tab):
    return _gather_kernel()(src, dst, p_tab, q_tab)


NPT = N // NS


@functools.cache
def _scatter_kernel():
    return functools.partial(
        pl.kernel,
        out_type=jax.ShapeDtypeStruct((NC, N, SW), jnp.float32),
        mesh=_mesh(),
        scratch_types=[
            pltpu.VMEM_SHARED((N, SW), jnp.float32),
            pltpu.VMEM((CH, LPR), jnp.int32),
            pltpu.VMEM((CH * LPR, SW), jnp.float32),
        ],
        compiler_params=pltpu.CompilerParams(use_tc_tiling_on_sc=False),
    )(_scatter_body)


def _scatter_body(scat_hbm, dst_hbm, zeros_hbm, nd_out, acc, idx_v, vals_v):
    cid = lax.axis_index("c")
    sid = lax.axis_index("s")
    w = sid * NC + cid

    @pl.when(sid == 0)
    def _():
        pltpu.sync_copy(zeros_hbm, acc)

    plsc.subcore_barrier()

    def chunk(jj, _):
        row0 = w * RPW + jj * CH

        @pl.when(row0 < ROWS)
        def _():
            pltpu.sync_copy(dst_hbm.at[pl.ds(row0, CH)], idx_v)
            pltpu.sync_copy(scat_hbm.at[pl.ds(row0 * LPR, CH * LPR)], vals_v)
            for j in range(CH):
                pltpu.sync_copy(vals_v.at[pl.ds(j * LPR, LPR)],
                                acc.at[idx_v.at[j]], add=True)

        return ()

    lax.fori_loop(0, RPW // CH, chunk, ())
    plsc.subcore_barrier()
    pltpu.sync_copy(acc.at[pl.ds(sid * NPT, NPT)],
                    nd_out.at[cid].at[pl.ds(sid * NPT, NPT)])


def _scatter_call(scat, dst, zeros_acc):
    return _scatter_kernel()(scat, dst, zeros_acc)


def _fold_conv(cp):
    we, wa, wm = cp['We'], cp['Wa'], cp['Wm']
    z = jnp.zeros
    wp = jnp.concatenate([we[:HID], wa[:HID], z((HID, 4), jnp.float32),
                          wm, z((HID, 8), jnp.float32)], axis=1)
    bp = jnp.concatenate([cp['be'], cp['ba'], z((4,), jnp.float32),
                          cp['bm'], z((8,), jnp.float32)])[None, :]
    wq = jnp.concatenate([we[HID:2 * HID], wa[HID:2 * HID],
                          z((HID, QW - 20), jnp.float32)], axis=1)
    return wp, bp, wq, we[2 * HID:], wa[2 * HID:]


def kernel(x, edge_index, edge_attr, params):
    p = params
    src = edge_index[0].reshape(ROWS, LPR)
    dst = edge_index[1].reshape(ROWS, LPR)
    zeros_acc = jnp.zeros((N, SW), jnp.float32)

    folded = [_fold_conv(p['conv0']), _fold_conv(p['conv1'])]
    r2 = lambda a: a[None, :]

    h, P, Q = _pre_call(
        x, r2(p['bn0_g']), r2(p['bn0_b']), p['W0'], r2(p['b0']),
        r2(p['bn1_g']), r2(p['bn1_b']), p['W1'], r2(p['b1']),
        folded[0][0], folded[0][1], folded[0][2])

    e_ev = edge_attr[0::2]
    e_od = edge_attr[1::2]
    scat2 = None
    for layer in range(5):
        wee, wae = folded[0 if layer == 0 else 1][3:]
        ps, qd = _gather_call(src, dst, P, Q)
        ps2 = ps.reshape(E2, 128)
        qd2 = qd.reshape(E2, 128)
        if layer == 0:
            scat2 = _edge0_call(ps2, qd2, e_ev, e_od, wee, wae)
        else:
            scat2 = _edge_call(ps2, qd2, scat2, wee, wae)
        nd = _scatter_call(scat2.reshape(E, SW), dst, zeros_acc)
        if layer < 4:
            nwp, nbp, nwq = folded[1][:3]
            h, P, Q = _update_call(h, nd, nwp, nbp, nwq)

    fc_args = []
    for i in range(NUM_FC):
        fc_args += [r2(p['fc%d_g' % i]), r2(p['fc%d_b' % i]),
                    p['fcW%d' % i], r2(p['fcb%d' % i])]
    return _final_call(h, nd, fc_args)


# speedup vs baseline: 57.3274x; 1.5014x over previous
import functools

import jax
import jax.numpy as jnp
from jax import lax
from jax.experimental import pallas as pl
from jax.experimental.pallas import tpu as pltpu
from jax.experimental.pallas import tpu_sc as plsc

N = 10000
E = 320000
D_IN = 128
HID = 32
D_EDGE = 16
HEADS = 4
HEAD_DIM = 8
FC = 64
NUM_FC = 5
EPS = 1e-5

PW = 64
QW = 64
SW = 64

NC, NS = 2, 16
NW = NC * NS
LPR = 128
ROWS = E // LPR
RPW = 80
RPAD = NW * RPW
CH = 5

_HIGH = jax.lax.Precision.DEFAULT


def _bn_relu(x, g, b):
    mu = jnp.mean(x, axis=0, keepdims=True)
    var = jnp.mean((x - mu) * (x - mu), axis=0, keepdims=True)
    return jnp.maximum((x - mu) / jnp.sqrt(var + EPS) * g + b, 0.0)


def _pre_body(x_ref, g0_ref, b0_ref, w0_ref, bb0_ref, g1_ref, b1_ref, w1_ref,
              bb1_ref, wp_ref, bp_ref, wq_ref, h_ref, p_ref, q_ref):
    x = jnp.nan_to_num(x_ref[...])
    h = jnp.dot(_bn_relu(x, g0_ref[...], b0_ref[...]), w0_ref[...],
                preferred_element_type=jnp.float32, precision=_HIGH) + bb0_ref[...]
    h = h + jnp.dot(_bn_relu(h, g1_ref[...], b1_ref[...]), w1_ref[...],
                    preferred_element_type=jnp.float32, precision=_HIGH) + bb1_ref[...]
    h_ref[...] = h
    p_ref[...] = jnp.dot(h, wp_ref[...], preferred_element_type=jnp.float32,
                         precision=_HIGH) + bp_ref[...]
    q_ref[...] = jnp.dot(h, wq_ref[...], preferred_element_type=jnp.float32,
                         precision=_HIGH)


def _pre_call(x, g0, b0, w0, bb0, g1, b1, w1, bb1, wp, bp, wq):
    return pl.pallas_call(
        _pre_body,
        out_shape=(
            jax.ShapeDtypeStruct((N, HID), jnp.float32),
            jax.ShapeDtypeStruct((N, PW), jnp.float32),
            jax.ShapeDtypeStruct((N, QW), jnp.float32),
        ),
    )(x, g0, b0, w0, bb0, g1, b1, w1, bb1, wp, bp, wq)


B2 = 4000
E2 = E // 2


def _edge_math(ps, qd, ev, wee, wae, rep):
    enew = jnp.maximum(
        ps[:, :16] + qd[:, :16]
        + jnp.dot(ev, wee, preferred_element_type=jnp.float32,
                  precision=_HIGH), 0.0)
    z = ps[:, 16:20] + qd[:, 16:20] + jnp.dot(
        enew, wae, preferred_element_type=jnp.float32, precision=_HIGH)
    z = jnp.where(z >= 0.0, z, 0.2 * z)
    ex = jnp.exp(z)
    hi = ex.astype(jnp.bfloat16).astype(jnp.float32)
    delta = ex - hi
    exb = (jnp.dot(hi, rep, preferred_element_type=jnp.float32)
           + jnp.dot(delta, rep, preferred_element_type=jnp.float32))
    msg = ps[:, 24:56] * exb
    zeros = jnp.zeros((ps.shape[0], 4), jnp.float32)
    return jnp.concatenate([msg, ex, zeros, enew, zeros, zeros], axis=1)


def _edge_body(ps_ref, qd_ref, ep_ref, wee_ref, wae_ref, rep_ref, out_ref):
    ps, qd, ep = ps_ref[...], qd_ref[...], ep_ref[...]
    cs = (wee_ref[...], wae_ref[...], rep_ref[...])
    out_l = _edge_math(ps[:, :64], qd[:, :64], ep[:, 40:56], *cs)
    out_r = _edge_math(ps[:, 64:], qd[:, 64:], ep[:, 104:120], *cs)
    out_ref[...] = jnp.concatenate([out_l, out_r], axis=1)


def _edge0_body(ps_ref, qd_ref, e2_ref, wee_ref, wae_ref, rep_ref, out_ref):
    ps, qd = ps_ref[...], qd_ref[...]
    e2 = jnp.nan_to_num(e2_ref[...])
    cs = (wee_ref[...], wae_ref[...], rep_ref[...])
    out_l = _edge_math(ps[:, :64], qd[:, :64], e2[:, :16], *cs)
    out_r = _edge_math(ps[:, 64:], qd[:, 64:], e2[:, 16:], *cs)
    out_ref[...] = jnp.concatenate([out_l, out_r], axis=1)


def _rep_const():
    rep = jnp.zeros((HEADS, HID), jnp.float32)
    for h in range(HEADS):
        rep = rep.at[h, 8 * h:8 * h + 8].set(1.0)
    return rep


_CONST_SPECS = [
    pl.BlockSpec((D_EDGE, D_EDGE), lambda i: (0, 0)),
    pl.BlockSpec((D_EDGE, HEADS), lambda i: (0, 0)),
    pl.BlockSpec((HEADS, HID), lambda i: (0, 0)),
]


def _edge_call(ps2, qd2, eprev2, wee, wae):
    return pl.pallas_call(
        _edge_body,
        grid=(E2 // B2,),
        in_specs=[
            pl.BlockSpec((B2, 128), lambda i: (i, 0)),
            pl.BlockSpec((B2, 128), lambda i: (i, 0)),
            pl.BlockSpec((B2, 128), lambda i: (i, 0)),
        ] + _CONST_SPECS,
        out_specs=pl.BlockSpec((B2, 128), lambda i: (i, 0)),
        out_shape=jax.ShapeDtypeStruct((E2, 128), jnp.float32),
    )(ps2, qd2, eprev2, wee, wae, _rep_const())


def _edge0_call(ps2, qd2, e2, wee, wae):
    return pl.pallas_call(
        _edge0_body,
        grid=(E2 // B2,),
        in_specs=[
            pl.BlockSpec((B2, 128), lambda i: (i, 0)),
            pl.BlockSpec((B2, 128), lambda i: (i, 0)),
            pl.BlockSpec((B2, 2 * D_EDGE), lambda i: (i, 0)),
        ] + _CONST_SPECS,
        out_specs=pl.BlockSpec((B2, 128), lambda i: (i, 0)),
        out_shape=jax.ShapeDtypeStruct((E2, 128), jnp.float32),
    )(ps2, qd2, e2, wee, wae, _rep_const())


def _agg(nd0, nd1):
    num = nd0[:, :HID] + nd1[:, :HID]
    den = nd0[:, HID:HID + HEADS] + nd1[:, HID:HID + HEADS]
    return jnp.concatenate(
        [num[:, h * 8:(h + 1) * 8] / (den[:, h:h + 1] + 1e-16)
         for h in range(HEADS)], axis=1)


def _update_body(h_ref, nd_ref, wp_ref, bp_ref, wq_ref, h_out, p_out, q_out):
    hn = jnp.maximum(h_ref[...] + _agg(nd_ref[0], nd_ref[1]), 0.0)
    h_out[...] = hn
    p_out[...] = jnp.dot(hn, wp_ref[...], preferred_element_type=jnp.float32,
                         precision=_HIGH) + bp_ref[...]
    q_out[...] = jnp.dot(hn, wq_ref[...], preferred_element_type=jnp.float32,
                         precision=_HIGH)


def _update_call(h, nd, wp, bp, wq):
    return pl.pallas_call(
        _update_body,
        out_shape=(
            jax.ShapeDtypeStruct((N, HID), jnp.float32),
            jax.ShapeDtypeStruct((N, PW), jnp.float32),
            jax.ShapeDtypeStruct((N, QW), jnp.float32),
        ),
    )(h, nd, wp, bp, wq)


def _final_body(h_ref, nd_ref, *refs):
    out_ref = refs[-1]
    h = jnp.maximum(h_ref[...] + _agg(nd_ref[0], nd_ref[1]), 0.0)
    for i in range(NUM_FC):
        g, b, w, fb = refs[4 * i:4 * i + 4]
        h = jnp.dot(_bn_relu(h, g[...], b[...]), w[...],
                    preferred_element_type=jnp.float32, precision=_HIGH) + fb[...]
    out_ref[...] = h


def _final_call(h, nd, fc_args):
    return pl.pallas_call(
        _final_body,
        out_shape=jax.ShapeDtypeStruct((N, FC), jnp.float32),
    )(h, nd, *fc_args)


@functools.cache
def _mesh():
    return plsc.VectorSubcoreMesh(core_axis_name="c", subcore_axis_name="s",
                                  num_cores=NC, num_subcores=NS)


@functools.cache
def _gather_kernel():
    return functools.partial(
        pl.kernel,
        out_type=(
            jax.ShapeDtypeStruct((E, PW), jnp.float32),
            jax.ShapeDtypeStruct((E, QW), jnp.float32),
        ),
        mesh=_mesh(),
        scratch_types=[
            pltpu.VMEM((RPW, LPR), jnp.int32),
            pltpu.VMEM((RPW, LPR), jnp.int32),
            pltpu.VMEM((CH * LPR, PW), jnp.float32),
            pltpu.VMEM((CH * LPR, QW), jnp.float32),
            pltpu.SemaphoreType.DMA,
        ],
        compiler_params=pltpu.CompilerParams(use_tc_tiling_on_sc=False),
    )(_gather_body)


def _gather_body(src_hbm, dst_hbm, p_hbm, q_hbm, ps_out, qd_out,
                 idxs_v, idxd_v, prow_v, qrow_v, sem):
    w = lax.axis_index("s") * NC + lax.axis_index("c")
    pltpu.sync_copy(src_hbm.at[pl.ds(w * RPW, RPW)], idxs_v)
    pltpu.sync_copy(dst_hbm.at[pl.ds(w * RPW, RPW)], idxd_v)

    def chunk(jj, _):
        row0 = w * RPW + jj * CH

        @pl.when(row0 < ROWS)
        def _():
            cps = []
            for j in range(CH):
                cps.append(pltpu.async_copy(
                    p_hbm.at[idxs_v.at[jj * CH + j]],
                    prow_v.at[pl.ds(j * LPR, LPR)], sem))
                cps.append(pltpu.async_copy(
                    q_hbm.at[idxd_v.at[jj * CH + j]],
                    qrow_v.at[pl.ds(j * LPR, LPR)], sem))
            for cp in cps:
                cp.wait()
            pltpu.sync_copy(prow_v, ps_out.at[pl.ds(row0 * LPR, CH * LPR)])
            pltpu.sync_copy(qrow_v, qd_out.at[pl.ds(row0 * LPR, CH * LPR)])

        return ()

    lax.fori_loop(0, RPW // CH, chunk, ())


def _gather_call(src, dst, p_tab, q_tab):
    return _gather_kernel()(src, dst, p_tab, q_tab)


NPT = N // NS


@functools.cache
def _scatter_kernel():
    return functools.partial(
        pl.kernel,
        out_type=jax.ShapeDtypeStruct((NC, N, SW), jnp.float32),
        mesh=_mesh(),
        scratch_types=[
            pltpu.VMEM_SHARED((N, SW), jnp.float32),
            pltpu.VMEM((RPW, LPR), jnp.int32),
            pltpu.VMEM((CH * LPR, SW), jnp.float32),
        ],
        compiler_params=pltpu.CompilerParams(use_tc_tiling_on_sc=False),
    )(_scatter_body)


def _scatter_body(scat_hbm, dst_hbm, zeros_hbm, nd_out, acc, idx_v, vals_v):
    cid = lax.axis_index("c")
    sid = lax.axis_index("s")
    w = sid * NC + cid

    @pl.when(sid == 0)
    def _():
        pltpu.sync_copy(zeros_hbm, acc)

    pltpu.sync_copy(dst_hbm.at[pl.ds(w * RPW, RPW)], idx_v)
    plsc.subcore_barrier()

    def chunk(jj, _):
        row0 = w * RPW + jj * CH

        @pl.when(row0 < ROWS)
        def _():
            pltpu.sync_copy(scat_hbm.at[pl.ds(row0 * LPR, CH * LPR)], vals_v)
            for j in range(CH):
                pltpu.sync_copy(vals_v.at[pl.ds(j * LPR, LPR)],
                                acc.at[idx_v.at[jj * CH + j]], add=True)

        return ()

    lax.fori_loop(0, RPW // CH, chunk, ())
    plsc.subcore_barrier()
    pltpu.sync_copy(acc.at[pl.ds(sid * NPT, NPT)],
                    nd_out.at[cid].at[pl.ds(sid * NPT, NPT)])


def _scatter_call(scat, dst, zeros_acc):
    return _scatter_kernel()(scat, dst, zeros_acc)


def _fold_conv(cp):
    we, wa, wm = cp['We'], cp['Wa'], cp['Wm']
    z = jnp.zeros
    wp = jnp.concatenate([we[:HID], wa[:HID], z((HID, 4), jnp.float32),
                          wm, z((HID, 8), jnp.float32)], axis=1)
    bp = jnp.concatenate([cp['be'], cp['ba'], z((4,), jnp.float32),
                          cp['bm'], z((8,), jnp.float32)])[None, :]
    wq = jnp.concatenate([we[HID:2 * HID], wa[HID:2 * HID],
                          z((HID, QW - 20), jnp.float32)], axis=1)
    return wp, bp, wq, we[2 * HID:], wa[2 * HID:]


def kernel(x, edge_index, edge_attr, params):
    p = params
    pad = jnp.zeros((RPAD - ROWS, LPR), jnp.int32)
    src = jnp.concatenate([edge_index[0].reshape(ROWS, LPR), pad])
    dst = jnp.concatenate([edge_index[1].reshape(ROWS, LPR), pad])
    zeros_acc = jnp.zeros((N, SW), jnp.float32)

    folded = [_fold_conv(p['conv0']), _fold_conv(p['conv1'])]
    r2 = lambda a: a[None, :]

    h, P, Q = _pre_call(
        x, r2(p['bn0_g']), r2(p['bn0_b']), p['W0'], r2(p['b0']),
        r2(p['bn1_g']), r2(p['bn1_b']), p['W1'], r2(p['b1']),
        folded[0][0], folded[0][1], folded[0][2])

    e2 = edge_attr.reshape(E2, 2 * D_EDGE)
    scat2 = None
    for layer in range(5):
        wee, wae = folded[0 if layer == 0 else 1][3:]
        ps, qd = _gather_call(src, dst, P, Q)
        ps2 = ps.reshape(E2, 128)
        qd2 = qd.reshape(E2, 128)
        if layer == 0:
            scat2 = _edge0_call(ps2, qd2, e2, wee, wae)
        else:
            scat2 = _edge_call(ps2, qd2, scat2, wee, wae)
        nd = _scatter_call(scat2.reshape(E, SW), dst, zeros_acc)
        if layer < 4:
            nwp, nbp, nwq = folded[1][:3]
            h, P, Q = _update_call(h, nd, nwp, nbp, nwq)

    fc_args = []
    for i in range(NUM_FC):
        fc_args += [r2(p['fc%d_g' % i]), r2(p['fc%d_b' % i]),
                    p['fcW%d' % i], r2(p['fcb%d' % i])]
    return _final_call(h, nd, fc_args)


# speedup vs baseline: 57.6536x; 1.0057x over previous
import functools

import jax
import jax.numpy as jnp
from jax import lax
from jax.experimental import pallas as pl
from jax.experimental.pallas import tpu as pltpu
from jax.experimental.pallas import tpu_sc as plsc

N = 10000
E = 320000
D_IN = 128
HID = 32
D_EDGE = 16
HEADS = 4
HEAD_DIM = 8
FC = 64
NUM_FC = 5
EPS = 1e-5

PW = 64
QW = 64
SW = 64

NC, NS = 2, 16
NW = NC * NS
LPR = 128
ROWS = E // LPR
RPW = 80
RPAD = NW * RPW
CH = 5

_HIGH = jax.lax.Precision.DEFAULT


def _bn_relu(x, g, b):
    mu = jnp.mean(x, axis=0, keepdims=True)
    var = jnp.mean((x - mu) * (x - mu), axis=0, keepdims=True)
    return jnp.maximum((x - mu) / jnp.sqrt(var + EPS) * g + b, 0.0)


def _pre_body(x_ref, g0_ref, b0_ref, w0_ref, bb0_ref, g1_ref, b1_ref, w1_ref,
              bb1_ref, wp_ref, bp_ref, wq_ref, h_ref, p_ref, q_ref):
    x = jnp.nan_to_num(x_ref[...])
    h = jnp.dot(_bn_relu(x, g0_ref[...], b0_ref[...]), w0_ref[...],
                preferred_element_type=jnp.float32, precision=_HIGH) + bb0_ref[...]
    h = h + jnp.dot(_bn_relu(h, g1_ref[...], b1_ref[...]), w1_ref[...],
                    preferred_element_type=jnp.float32, precision=_HIGH) + bb1_ref[...]
    h_ref[...] = h
    p_ref[...] = jnp.dot(h, wp_ref[...], preferred_element_type=jnp.float32,
                         precision=_HIGH) + bp_ref[...]
    q_ref[...] = jnp.dot(h, wq_ref[...], preferred_element_type=jnp.float32,
                         precision=_HIGH)


def _pre_call(x, g0, b0, w0, bb0, g1, b1, w1, bb1, wp, bp, wq):
    return pl.pallas_call(
        _pre_body,
        out_shape=(
            jax.ShapeDtypeStruct((N, HID), jnp.float32),
            jax.ShapeDtypeStruct((N, PW), jnp.float32),
            jax.ShapeDtypeStruct((N, QW), jnp.float32),
        ),
    )(x, g0, b0, w0, bb0, g1, b1, w1, bb1, wp, bp, wq)


B2 = 8000
E2 = E // 2


def _edge_math(ps, qd, ev, wee, wae, rep):
    enew = jnp.maximum(
        ps[:, :16] + qd[:, :16]
        + jnp.dot(ev, wee, preferred_element_type=jnp.float32,
                  precision=_HIGH), 0.0)
    z = ps[:, 16:20] + qd[:, 16:20] + jnp.dot(
        enew, wae, preferred_element_type=jnp.float32, precision=_HIGH)
    z = jnp.where(z >= 0.0, z, 0.2 * z)
    ex = jnp.exp(z)
    hi = ex.astype(jnp.bfloat16).astype(jnp.float32)
    delta = ex - hi
    exb = (jnp.dot(hi, rep, preferred_element_type=jnp.float32)
           + jnp.dot(delta, rep, preferred_element_type=jnp.float32))
    msg = ps[:, 24:56] * exb
    zeros = jnp.zeros((ps.shape[0], 4), jnp.float32)
    return jnp.concatenate([msg, ex, zeros, enew, zeros, zeros], axis=1)


def _edge_body(ps_ref, qd_ref, ep_ref, wee_ref, wae_ref, rep_ref, out_ref):
    ps, qd, ep = ps_ref[...], qd_ref[...], ep_ref[...]
    cs = (wee_ref[...], wae_ref[...], rep_ref[...])
    out_l = _edge_math(ps[:, :64], qd[:, :64], ep[:, 40:56], *cs)
    out_r = _edge_math(ps[:, 64:], qd[:, 64:], ep[:, 104:120], *cs)
    out_ref[...] = jnp.concatenate([out_l, out_r], axis=1)


def _edge0_body(ps_ref, qd_ref, e2_ref, wee_ref, wae_ref, rep_ref, out_ref):
    ps, qd = ps_ref[...], qd_ref[...]
    e2 = jnp.nan_to_num(e2_ref[...])
    cs = (wee_ref[...], wae_ref[...], rep_ref[...])
    out_l = _edge_math(ps[:, :64], qd[:, :64], e2[:, :16], *cs)
    out_r = _edge_math(ps[:, 64:], qd[:, 64:], e2[:, 16:], *cs)
    out_ref[...] = jnp.concatenate([out_l, out_r], axis=1)


def _rep_const():
    rep = jnp.zeros((HEADS, HID), jnp.float32)
    for h in range(HEADS):
        rep = rep.at[h, 8 * h:8 * h + 8].set(1.0)
    return rep


_CONST_SPECS = [
    pl.BlockSpec((D_EDGE, D_EDGE), lambda i: (0, 0)),
    pl.BlockSpec((D_EDGE, HEADS), lambda i: (0, 0)),
    pl.BlockSpec((HEADS, HID), lambda i: (0, 0)),
]


def _edge_call(ps2, qd2, eprev2, wee, wae):
    return pl.pallas_call(
        _edge_body,
        grid=(E2 // B2,),
        in_specs=[
            pl.BlockSpec((B2, 128), lambda i: (i, 0)),
            pl.BlockSpec((B2, 128), lambda i: (i, 0)),
            pl.BlockSpec((B2, 128), lambda i: (i, 0)),
        ] + _CONST_SPECS,
        out_specs=pl.BlockSpec((B2, 128), lambda i: (i, 0)),
        out_shape=jax.ShapeDtypeStruct((E2, 128), jnp.float32),
    )(ps2, qd2, eprev2, wee, wae, _rep_const())


def _edge0_call(ps2, qd2, e2, wee, wae):
    return pl.pallas_call(
        _edge0_body,
        grid=(E2 // B2,),
        in_specs=[
            pl.BlockSpec((B2, 128), lambda i: (i, 0)),
            pl.BlockSpec((B2, 128), lambda i: (i, 0)),
            pl.BlockSpec((B2, 2 * D_EDGE), lambda i: (i, 0)),
        ] + _CONST_SPECS,
        out_specs=pl.BlockSpec((B2, 128), lambda i: (i, 0)),
        out_shape=jax.ShapeDtypeStruct((E2, 128), jnp.float32),
    )(ps2, qd2, e2, wee, wae, _rep_const())


def _agg(nd0, nd1):
    num = nd0[:, :HID] + nd1[:, :HID]
    den = nd0[:, HID:HID + HEADS] + nd1[:, HID:HID + HEADS]
    return jnp.concatenate(
        [num[:, h * 8:(h + 1) * 8] / (den[:, h:h + 1] + 1e-16)
         for h in range(HEADS)], axis=1)


def _update_body(h_ref, nd_ref, wp_ref, bp_ref, wq_ref, h_out, p_out, q_out):
    hn = jnp.maximum(h_ref[...] + _agg(nd_ref[0], nd_ref[1]), 0.0)
    h_out[...] = hn
    p_out[...] = jnp.dot(hn, wp_ref[...], preferred_element_type=jnp.float32,
                         precision=_HIGH) + bp_ref[...]
    q_out[...] = jnp.dot(hn, wq_ref[...], preferred_element_type=jnp.float32,
                         precision=_HIGH)


def _update_call(h, nd, wp, bp, wq):
    return pl.pallas_call(
        _update_body,
        out_shape=(
            jax.ShapeDtypeStruct((N, HID), jnp.float32),
            jax.ShapeDtypeStruct((N, PW), jnp.float32),
            jax.ShapeDtypeStruct((N, QW), jnp.float32),
        ),
    )(h, nd, wp, bp, wq)


def _final_body(h_ref, nd_ref, *refs):
    out_ref = refs[-1]
    h = jnp.maximum(h_ref[...] + _agg(nd_ref[0], nd_ref[1]), 0.0)
    for i in range(NUM_FC):
        g, b, w, fb = refs[4 * i:4 * i + 4]
        h = jnp.dot(_bn_relu(h, g[...], b[...]), w[...],
                    preferred_element_type=jnp.float32, precision=_HIGH) + fb[...]
    out_ref[...] = h


def _final_call(h, nd, fc_args):
    return pl.pallas_call(
        _final_body,
        out_shape=jax.ShapeDtypeStruct((N, FC), jnp.float32),
    )(h, nd, *fc_args)


@functools.cache
def _mesh():
    return plsc.VectorSubcoreMesh(core_axis_name="c", subcore_axis_name="s",
                                  num_cores=NC, num_subcores=NS)


@functools.cache
def _gather_kernel():
    return functools.partial(
        pl.kernel,
        out_type=(
            jax.ShapeDtypeStruct((E, PW), jnp.float32),
            jax.ShapeDtypeStruct((E, QW), jnp.float32),
        ),
        mesh=_mesh(),
        scratch_types=[
            pltpu.VMEM((RPW, LPR), jnp.int32),
            pltpu.VMEM((RPW, LPR), jnp.int32),
            pltpu.VMEM((CH * LPR, PW), jnp.float32),
            pltpu.VMEM((CH * LPR, QW), jnp.float32),
            pltpu.SemaphoreType.DMA,
        ],
        compiler_params=pltpu.CompilerParams(use_tc_tiling_on_sc=False),
    )(_gather_body)


def _gather_body(src_hbm, dst_hbm, p_hbm, q_hbm, ps_out, qd_out,
                 idxs_v, idxd_v, prow_v, qrow_v, sem):
    w = lax.axis_index("s") * NC + lax.axis_index("c")
    pltpu.sync_copy(src_hbm.at[pl.ds(w * RPW, RPW)], idxs_v)
    pltpu.sync_copy(dst_hbm.at[pl.ds(w * RPW, RPW)], idxd_v)

    def chunk(jj, _):
        row0 = w * RPW + jj * CH

        @pl.when(row0 < ROWS)
        def _():
            cps = []
            for j in range(CH):
                cps.append(pltpu.async_copy(
                    p_hbm.at[idxs_v.at[jj * CH + j]],
                    prow_v.at[pl.ds(j * LPR, LPR)], sem))
                cps.append(pltpu.async_copy(
                    q_hbm.at[idxd_v.at[jj * CH + j]],
                    qrow_v.at[pl.ds(j * LPR, LPR)], sem))
            for cp in cps:
                cp.wait()
            pltpu.sync_copy(prow_v, ps_out.at[pl.ds(row0 * LPR, CH * LPR)])
            pltpu.sync_copy(qrow_v, qd_out.at[pl.ds(row0 * LPR, CH * LPR)])

        return ()

    lax.fori_loop(0, RPW // CH, chunk, ())


def _gather_call(src, dst, p_tab, q_tab):
    return _gather_kernel()(src, dst, p_tab, q_tab)


NPT = N // NS


@functools.cache
def _scatter_kernel():
    return functools.partial(
        pl.kernel,
        out_type=jax.ShapeDtypeStruct((NC, N, SW), jnp.float32),
        mesh=_mesh(),
        scratch_types=[
            pltpu.VMEM_SHARED((N, SW), jnp.float32),
            pltpu.VMEM((RPW, LPR), jnp.int32),
            pltpu.VMEM((CH * LPR, SW), jnp.float32),
        ],
        compiler_params=pltpu.CompilerParams(use_tc_tiling_on_sc=False),
    )(_scatter_body)


def _scatter_body(scat_hbm, dst_hbm, zeros_hbm, nd_out, acc, idx_v, vals_v):
    cid = lax.axis_index("c")
    sid = lax.axis_index("s")
    w = sid * NC + cid

    @pl.when(sid == 0)
    def _():
        pltpu.sync_copy(zeros_hbm, acc)

    pltpu.sync_copy(dst_hbm.at[pl.ds(w * RPW, RPW)], idx_v)
    plsc.subcore_barrier()

    def chunk(jj, _):
        row0 = w * RPW + jj * CH

        @pl.when(row0 < ROWS)
        def _():
            pltpu.sync_copy(scat_hbm.at[pl.ds(row0 * LPR, CH * LPR)], vals_v)
            for j in range(CH):
                pltpu.sync_copy(vals_v.at[pl.ds(j * LPR, LPR)],
                                acc.at[idx_v.at[jj * CH + j]], add=True)

        return ()

    lax.fori_loop(0, RPW // CH, chunk, ())
    plsc.subcore_barrier()
    pltpu.sync_copy(acc.at[pl.ds(sid * NPT, NPT)],
                    nd_out.at[cid].at[pl.ds(sid * NPT, NPT)])


def _scatter_call(scat, dst, zeros_acc):
    return _scatter_kernel()(scat, dst, zeros_acc)


def _fold_conv(cp):
    we, wa, wm = cp['We'], cp['Wa'], cp['Wm']
    z = jnp.zeros
    wp = jnp.concatenate([we[:HID], wa[:HID], z((HID, 4), jnp.float32),
                          wm, z((HID, 8), jnp.float32)], axis=1)
    bp = jnp.concatenate([cp['be'], cp['ba'], z((4,), jnp.float32),
                          cp['bm'], z((8,), jnp.float32)])[None, :]
    wq = jnp.concatenate([we[HID:2 * HID], wa[HID:2 * HID],
                          z((HID, QW - 20), jnp.float32)], axis=1)
    return wp, bp, wq, we[2 * HID:], wa[2 * HID:]


def kernel(x, edge_index, edge_attr, params):
    p = params
    pad = jnp.zeros((RPAD - ROWS, LPR), jnp.int32)
    src = jnp.concatenate([edge_index[0].reshape(ROWS, LPR), pad])
    dst = jnp.concatenate([edge_index[1].reshape(ROWS, LPR), pad])
    zeros_acc = jnp.zeros((N, SW), jnp.float32)

    folded = [_fold_conv(p['conv0']), _fold_conv(p['conv1'])]
    r2 = lambda a: a[None, :]

    h, P, Q = _pre_call(
        x, r2(p['bn0_g']), r2(p['bn0_b']), p['W0'], r2(p['b0']),
        r2(p['bn1_g']), r2(p['bn1_b']), p['W1'], r2(p['b1']),
        folded[0][0], folded[0][1], folded[0][2])

    e2 = edge_attr.reshape(E2, 2 * D_EDGE)
    scat2 = None
    for layer in range(5):
        wee, wae = folded[0 if layer == 0 else 1][3:]
        ps, qd = _gather_call(src, dst, P, Q)
        ps2 = ps.reshape(E2, 128)
        qd2 = qd.reshape(E2, 128)
        if layer == 0:
            scat2 = _edge0_call(ps2, qd2, e2, wee, wae)
        else:
            scat2 = _edge_call(ps2, qd2, scat2, wee, wae)
        nd = _scatter_call(scat2.reshape(E, SW), dst, zeros_acc)
        if layer < 4:
            nwp, nbp, nwq = folded[1][:3]
            h, P, Q = _update_call(h, nd, nwp, nbp, nwq)

    fc_args = []
    for i in range(NUM_FC):
        fc_args += [r2(p['fc%d_g' % i]), r2(p['fc%d_b' % i]),
                    p['fcW%d' % i], r2(p['fcb%d' % i])]
    return _final_call(h, nd, fc_args)
